# baseline (speedup 1.0000x reference)
import jax
import jax.numpy as jnp
from jax import lax
from jax.experimental import pallas as pl
from jax.experimental.pallas import tpu as pltpu
from jax.experimental.pallas import tpu_sc as plsc

EPS = 1e-5
LANES = 16
NC = 2
NS = 16
NW = NC * NS
CHUNK = 128


def _tec_body(word_hbm, src_hbm, pos_hbm, gamma_hbm, beta_hbm, out_hbm,
              idx_v, rows_v, out_v, pos_v, gb_v, sem):
    n_chunks, chunk_r = idx_v.shape
    rpw, hidden = rows_v.shape
    ppw = pos_v.shape[0]
    batch = rpw // ppw
    nvec = hidden // LANES

    wid = lax.axis_index("s") * NC + lax.axis_index("c")

    pltpu.sync_copy(src_hbm.at[pl.ds(wid * n_chunks, n_chunks)], idx_v)
    copies = [
        pltpu.async_copy(word_hbm.at[idx_v.at[j]],
                         rows_v.at[pl.ds(j * chunk_r, chunk_r)], sem)
        for j in range(n_chunks)
    ]
    pltpu.sync_copy(pos_hbm.at[pl.ds(wid * ppw, ppw)], pos_v)
    pltpu.sync_copy(gamma_hbm, gb_v.at[0])
    pltpu.sync_copy(beta_hbm, gb_v.at[1])
    for c in copies:
        c.wait()

    g = [gb_v[0, pl.ds(LANES * i, LANES)] for i in range(nvec)]
    bt = [gb_v[1, pl.ds(LANES * i, LANES)] for i in range(nvec)]
    inv_h = jnp.float32(1.0 / hidden)
    lane = lax.iota(jnp.int32, LANES)
    perms = [lane ^ (1 << k) for k in range(4)]
    dnums = lax.GatherDimensionNumbers(
        offset_dims=(), collapsed_slice_dims=(0,), start_index_map=(0,))

    def allsum(v):
        for p in perms:
            v = v + lax.gather(v, p[:, None], dimension_numbers=dnums,
                               slice_sizes=(1,),
                               mode=lax.GatherScatterMode.PROMISE_IN_BOUNDS)
        return v

    @plsc.parallel_loop(0, ppw, unroll=1)
    def _pos_loop(p):
        pos_regs = [pos_v[p, pl.ds(LANES * i, LANES)] for i in range(nvec)]
        for b in range(batch):
            r = p * batch + b
            x = [rows_v[r, pl.ds(LANES * i, LANES)] + pos_regs[i]
                 for i in range(nvec)]
            t = x
            while len(t) > 1:
                t = [t[2 * i] + t[2 * i + 1] for i in range(len(t) // 2)]
            sq = [xi * xi for xi in x]
            while len(sq) > 1:
                sq = [sq[2 * i] + sq[2 * i + 1] for i in range(len(sq) // 2)]
            mu_v = allsum(t[0]) * inv_h
            var_v = allsum(sq[0]) * inv_h - mu_v * mu_v
            vv = var_v + EPS
            ii = lax.bitcast_convert_type(vv, jnp.int32)
            y = lax.bitcast_convert_type(
                jnp.int32(0x5F3759DF) - (ii >> 1), jnp.float32)
            for _newton in range(2):
                y = y * (1.5 - 0.5 * vv * y * y)
            for i in range(nvec):
                out_v[r, pl.ds(LANES * i, LANES)] = (
                    (x[i] - mu_v) * y * g[i] + bt[i])

    pltpu.sync_copy(out_v, out_hbm.at[pl.ds(wid * rpw, rpw)])


def kernel(src, word_table, pos_table, gamma, beta):
    S, B = src.shape
    H = word_table.shape[1]
    rows = S * B
    rpw = rows // NW
    ppw = S // NW

    src2d = src.reshape(NW * 2, rpw // 2)

    mesh = plsc.VectorSubcoreMesh(core_axis_name="c", subcore_axis_name="s")
    k = pl.kernel(
        _tec_body,
        mesh=mesh,
        out_type=jax.ShapeDtypeStruct((rows, H), jnp.float32),
        scratch_types=[
            pltpu.VMEM((2, rpw // 2), jnp.int32),
            pltpu.VMEM((rpw, H), jnp.float32),
            pltpu.VMEM((rpw, H), jnp.float32),
            pltpu.VMEM((ppw, H), jnp.float32),
            pltpu.VMEM((2, H), jnp.float32),
            pltpu.SemaphoreType.DMA,
        ],
    )
    out = k(word_table, src2d, pos_table, gamma, beta)
    return out.reshape(S, B, H)


# speedup vs baseline: 1.0381x; 1.0381x over previous
import jax
import jax.numpy as jnp
from jax import lax
from jax.experimental import pallas as pl
from jax.experimental.pallas import tpu as pltpu
from jax.experimental.pallas import tpu_sc as plsc

EPS = 1e-5
LANES = 16
NC = 2
NS = 16
NW = NC * NS
CHUNK = 128


def _tec_body(word_hbm, src_hbm, pos_hbm, gamma_hbm, beta_hbm, out_hbm,
              idx_v, rows_v, out_v, pos_v, gb_v, sem):
    n_chunks, chunk_r = idx_v.shape
    rpw, hidden = rows_v.shape
    ppw = pos_v.shape[0]
    batch = rpw // ppw
    nvec = hidden // LANES

    wid = lax.axis_index("s") * NC + lax.axis_index("c")

    pltpu.sync_copy(src_hbm.at[pl.ds(wid * n_chunks, n_chunks)], idx_v)
    copies = [
        pltpu.async_copy(word_hbm.at[idx_v.at[j]],
                         rows_v.at[pl.ds(j * chunk_r, chunk_r)], sem)
        for j in range(n_chunks)
    ]
    pltpu.sync_copy(pos_hbm.at[pl.ds(wid * ppw, ppw)], pos_v)
    pltpu.sync_copy(gamma_hbm, gb_v.at[0])
    pltpu.sync_copy(beta_hbm, gb_v.at[1])
    for c in copies:
        c.wait()

    g = [gb_v[0, pl.ds(LANES * i, LANES)] for i in range(nvec)]
    bt = [gb_v[1, pl.ds(LANES * i, LANES)] for i in range(nvec)]
    inv_h = jnp.float32(1.0 / hidden)
    lane = lax.iota(jnp.int32, LANES)
    perms = [lane ^ (1 << k) for k in range(4)]
    dnums = lax.GatherDimensionNumbers(
        offset_dims=(), collapsed_slice_dims=(0,), start_index_map=(0,))

    def allsum(v):
        for p in perms:
            v = v + lax.gather(v, p[:, None], dimension_numbers=dnums,
                               slice_sizes=(1,),
                               mode=lax.GatherScatterMode.PROMISE_IN_BOUNDS)
        return v

    log2b = batch.bit_length() - 1 if batch & (batch - 1) == 0 else None

    @plsc.parallel_loop(0, rpw, unroll=2)
    def _row_loop(r):
        p = (r >> log2b) if log2b is not None else lax.div(r, batch)
        x = [rows_v[r, pl.ds(LANES * i, LANES)]
             + pos_v[p, pl.ds(LANES * i, LANES)] for i in range(nvec)]
        t = x
        while len(t) > 1:
            t = [t[2 * i] + t[2 * i + 1] for i in range(len(t) // 2)]
        sq = [xi * xi for xi in x]
        while len(sq) > 1:
            sq = [sq[2 * i] + sq[2 * i + 1] for i in range(len(sq) // 2)]
        mu_v = allsum(t[0]) * inv_h
        var_v = allsum(sq[0]) * inv_h - mu_v * mu_v
        vv = var_v + EPS
        ii = lax.bitcast_convert_type(vv, jnp.int32)
        y = lax.bitcast_convert_type(
            jnp.int32(0x5F3759DF) - (ii >> 1), jnp.float32)
        for _newton in range(2):
            y = y * (1.5 - 0.5 * vv * y * y)
        for i in range(nvec):
            out_v[r, pl.ds(LANES * i, LANES)] = (
                (x[i] - mu_v) * y * g[i] + bt[i])

    pltpu.sync_copy(out_v, out_hbm.at[pl.ds(wid * rpw, rpw)])


def kernel(src, word_table, pos_table, gamma, beta):
    S, B = src.shape
    H = word_table.shape[1]
    rows = S * B
    rpw = rows // NW
    ppw = S // NW

    src2d = src.reshape(NW * 2, rpw // 2)

    mesh = plsc.VectorSubcoreMesh(core_axis_name="c", subcore_axis_name="s")
    k = pl.kernel(
        _tec_body,
        mesh=mesh,
        out_type=jax.ShapeDtypeStruct((rows, H), jnp.float32),
        scratch_types=[
            pltpu.VMEM((2, rpw // 2), jnp.int32),
            pltpu.VMEM((rpw, H), jnp.float32),
            pltpu.VMEM((rpw, H), jnp.float32),
            pltpu.VMEM((ppw, H), jnp.float32),
            pltpu.VMEM((2, H), jnp.float32),
            pltpu.SemaphoreType.DMA,
        ],
    )
    out = k(word_table, src2d, pos_table, gamma, beta)
    return out.reshape(S, B, H)
